# baseline (speedup 1.0000x reference)
import functools

import jax
import jax.numpy as jnp
from jax import lax
from jax.experimental import pallas as pl
from jax.experimental.pallas import tpu as pltpu
from jax.experimental.pallas import tpu_sc as plsc

_D = 64
_SCALE = 8.0
_NC = 2
_NS = 16
_NW = _NC * _NS
_BW = 128
_K = 128
_NBUF = 5
_PF = 3


@functools.lru_cache(maxsize=None)
def _build(batch, seq):
    assert batch == _BW * _NW
    n_chunks = _BW * seq // _K
    mesh = plsc.VectorSubcoreMesh(core_axis_name="c", subcore_axis_name="s")

    @functools.partial(
        pl.kernel,
        mesh=mesh,
        out_type=jax.ShapeDtypeStruct((batch * seq, 128), jnp.float32),
        compiler_params=pltpu.CompilerParams(needs_layout_passes=False),
        scratch_types=[
            pltpu.VMEM((_BW * seq,), jnp.int32),
            pltpu.VMEM((_NBUF, _K, 128), jnp.float32),
            pltpu.SemaphoreType.DMA((_NBUF,)),
            pltpu.SemaphoreType.DMA((_NBUF,)),
        ],
    )
    def gather_scale(idx_hbm, table_hbm, out_hbm, xb_v, rows_v, g_sem, s_sem):
        wid = lax.axis_index("s") * _NC + lax.axis_index("c")
        b0 = wid * _BW

        pltpu.sync_copy(idx_hbm.at[pl.ds(b0 * seq, _BW * seq)], xb_v)

        def gather_start(c, b):
            pltpu.async_copy(table_hbm.at[xb_v.at[pl.ds(c * _K, _K)]],
                             rows_v.at[b], g_sem.at[b])

        def gather_wait(b):
            pltpu.make_async_copy(table_hbm.at[xb_v.at[pl.ds(0, _K)]],
                                  rows_v.at[b], g_sem.at[b]).wait()

        def scatter_start(c, b):
            pltpu.async_copy(rows_v.at[b],
                             out_hbm.at[pl.ds(b0 * seq + c * _K, _K)],
                             s_sem.at[b])

        def scatter_wait(b):
            pltpu.make_async_copy(rows_v.at[b], out_hbm.at[pl.ds(0, _K)],
                                  s_sem.at[b]).wait()

        for b in range(_PF):
            gather_start(b, b)

        def group(g, carry):
            for b in range(_NBUF):
                c = g * _NBUF + b
                gather_wait(b)

                def sc_body(r0, c2):
                    for ur in range(4):
                        r = r0 * 4 + ur
                        for u in range(_D // 16):
                            rows_v[b, r, pl.ds(u * 16, 16)] = (
                                rows_v[b, r, pl.ds(u * 16, 16)] * _SCALE
                            )
                    return c2

                lax.fori_loop(0, _K // 4, sc_body, 0)
                scatter_start(c, b)

                cp = c + _PF
                bp = (b + _PF) % _NBUF

                @pl.when(cp < n_chunks)
                def _():
                    @pl.when(cp >= _NBUF)
                    def _():
                        scatter_wait(bp)

                    gather_start(cp, bp)

            return carry

        lax.fori_loop(0, n_chunks // _NBUF, group, 0)

        for b in range(_NBUF):
            scatter_wait(b)

    return gather_scale


def kernel(x, embedding):
    batch, seq = x.shape
    idx = x.reshape(-1).astype(jnp.int32)
    tab_p = jnp.pad(embedding, ((0, 0), (0, 128 - _D)))
    out_p = _build(batch, seq)(idx, tab_p)
    return out_p.reshape(batch, seq, 128)[:, :, :_D]


# speedup vs baseline: 1.0010x; 1.0010x over previous
import functools

import jax
import jax.numpy as jnp
from jax import lax
from jax.experimental import pallas as pl
from jax.experimental.pallas import tpu as pltpu
from jax.experimental.pallas import tpu_sc as plsc

_D = 64
_SCALE = 8.0
_NC = 2
_NS = 16
_NW = _NC * _NS
_BW = 128
_K = 128
_NBUF = 4
_PF = 2


@functools.lru_cache(maxsize=None)
def _build(batch, seq):
    assert batch == _BW * _NW
    n_chunks = _BW * seq // _K
    mesh = plsc.VectorSubcoreMesh(core_axis_name="c", subcore_axis_name="s")

    @functools.partial(
        pl.kernel,
        mesh=mesh,
        out_type=jax.ShapeDtypeStruct((batch * seq, 128), jnp.float32),
        compiler_params=pltpu.CompilerParams(needs_layout_passes=False),
        scratch_types=[
            pltpu.VMEM((_BW * seq,), jnp.int32),
            pltpu.VMEM((_NBUF, _K, 128), jnp.float32),
            pltpu.SemaphoreType.DMA((_NBUF,)),
            pltpu.SemaphoreType.DMA((_NBUF,)),
        ],
    )
    def gather_scale(idx_hbm, table_hbm, out_hbm, xb_v, rows_v, g_sem, s_sem):
        wid = lax.axis_index("s") * _NC + lax.axis_index("c")
        b0 = wid * _BW

        pltpu.sync_copy(idx_hbm.at[pl.ds(b0 * seq, _BW * seq)], xb_v)

        def gather_start(c, b):
            pltpu.async_copy(table_hbm.at[xb_v.at[pl.ds(c * _K, _K)]],
                             rows_v.at[b], g_sem.at[b])

        def gather_wait(b):
            pltpu.make_async_copy(table_hbm.at[xb_v.at[pl.ds(0, _K)]],
                                  rows_v.at[b], g_sem.at[b]).wait()

        def scatter_start(c, b):
            pltpu.async_copy(rows_v.at[b],
                             out_hbm.at[pl.ds(b0 * seq + c * _K, _K)],
                             s_sem.at[b])

        def scatter_wait(b):
            pltpu.make_async_copy(rows_v.at[b], out_hbm.at[pl.ds(0, _K)],
                                  s_sem.at[b]).wait()

        for b in range(_PF):
            gather_start(b, b)

        def group(g, carry):
            for b in range(_NBUF):
                c = g * _NBUF + b
                gather_wait(b)

                def sc_body(r0, c2):
                    for ur in range(4):
                        r = r0 * 4 + ur
                        for u in range(_D // 16):
                            rows_v[b, r, pl.ds(u * 16, 16)] = (
                                rows_v[b, r, pl.ds(u * 16, 16)] * _SCALE
                            )
                    return c2

                lax.fori_loop(0, _K // 4, sc_body, 0)
                scatter_start(c, b)

                cp = c + _PF
                bp = (b + _PF) % _NBUF

                @pl.when(cp < n_chunks)
                def _():
                    @pl.when(cp >= _NBUF)
                    def _():
                        scatter_wait(bp)

                    gather_start(cp, bp)

            return carry

        lax.fori_loop(0, n_chunks // _NBUF, group, 0)

        for b in range(_NBUF):
            scatter_wait(b)

    return gather_scale


def kernel(x, embedding):
    batch, seq = x.shape
    idx = x.reshape(-1).astype(jnp.int32)
    tab_p = jnp.pad(embedding, ((0, 0), (0, 128 - _D)))
    out_p = _build(batch, seq)(idx, tab_p)
    return out_p.reshape(batch, seq, 128)[:, :, :_D]
